# baseline (speedup 1.0000x reference)
import functools

import jax
import jax.numpy as jnp
import numpy as np
from jax.experimental import pallas as pl

_K = 64
_W = 64
_INT_MIN = np.int32(-(2 ** 31))
_TOP_MASK = np.int32(0x7FFFFFFF)


def _sortable(x):
    xi = jax.lax.bitcast_convert_type(x, jnp.int32)
    return xi ^ (jax.lax.shift_right_arithmetic(xi, 31) & _TOP_MASK)


def _radix_kth(s, k, axes):
    def body(i, p):
        bit = jax.lax.shift_left(np.int32(1), (31 - i).astype(jnp.int32))
        cand = p | bit
        thr = cand ^ _INT_MIN
        cnt = jnp.sum((s >= thr).astype(jnp.int32), axis=axes, keepdims=True)
        return jnp.where(cnt >= k, cand, p)

    shape = tuple(1 if d in axes else n for d, n in enumerate(s.shape))
    p = jax.lax.fori_loop(0, 32, body, jnp.zeros(shape, jnp.int32), unroll=True)
    return p ^ _INT_MIN


def _topk_mask_kernel(x_ref, o_ref, *, k):
    x = x_ref[...]
    r, w, g = x.shape
    s = _sortable(x)

    sg = jnp.max(s, axis=1)
    vg = _radix_kth(sg, k, axes=(1,))

    gt = sg > vg
    eq = sg == vg
    gtf = gt.astype(jnp.float32)
    eqf = eq.astype(jnp.float32)
    ia = jax.lax.broadcasted_iota(jnp.int32, (g, g), 0)
    ib = jax.lax.broadcasted_iota(jnp.int32, (g, g), 1)
    tri = (ia <= ib).astype(jnp.float32)
    cum_gt = jnp.dot(gtf, tri, preferred_element_type=jnp.float32)
    cum_eq = jnp.dot(eqf, tri, preferred_element_type=jnp.float32)
    cgt = jnp.sum(gtf, axis=1, keepdims=True)
    rgt = cum_gt - gtf
    req = cum_eq - eqf + cgt
    rank = jnp.where(gt, rgt, jnp.where(eq, req, np.float32(1e9)))

    slots = jax.lax.broadcasted_iota(jnp.int32, (1, k, 1), 1).astype(jnp.float32)
    sel = (rank[:, None, :] == slots).astype(jnp.float32)
    compact = jax.lax.dot_general(
        sel, x, (((2,), (2,)), ((0,), (0,))),
        precision=jax.lax.Precision.HIGHEST,
        preferred_element_type=jnp.float32)

    v = _radix_kth(_sortable(compact), k, axes=(1, 2))

    o_ref[...] = jnp.where(s >= v, jnp.maximum(x, 0.0), 0.0)


def _topk_mask_3d(x3, k, rows_per_block):
    n_rows, w, g = x3.shape
    body = functools.partial(_topk_mask_kernel, k=k)
    return pl.pallas_call(
        body,
        grid=(n_rows // rows_per_block,),
        in_specs=[pl.BlockSpec((rows_per_block, w, g), lambda i: (i, 0, 0))],
        out_specs=pl.BlockSpec((rows_per_block, w, g), lambda i: (i, 0, 0)),
        out_shape=jax.ShapeDtypeStruct((n_rows, w, g), x3.dtype),
    )(x3)


def kernel(features):
    b, l, d = features.shape
    x3 = features.reshape(b * l, _W, d // _W)
    out = _topk_mask_3d(x3, _K, 64)
    return out.reshape(b, l, d)


# speedup vs baseline: 1.0003x; 1.0003x over previous
import functools

import jax
import jax.numpy as jnp
import numpy as np
from jax.experimental import pallas as pl

_K = 64
_W = 64
_INT_MIN = np.int32(-(2 ** 31))
_TOP_MASK = np.int32(0x7FFFFFFF)


def _sortable(x):
    xi = jax.lax.bitcast_convert_type(x, jnp.int32)
    return xi ^ (jax.lax.shift_right_arithmetic(xi, 31) & _TOP_MASK)


def _radix_kth(s, k, axes):
    def body(i, p):
        bit = jax.lax.shift_left(np.int32(1), (31 - i).astype(jnp.int32))
        cand = p | bit
        thr = cand ^ _INT_MIN
        cnt = jnp.sum((s >= thr).astype(jnp.int32), axis=axes, keepdims=True)
        return jnp.where(cnt >= k, cand, p)

    shape = tuple(1 if d in axes else n for d, n in enumerate(s.shape))
    p = jax.lax.fori_loop(0, 32, body, jnp.zeros(shape, jnp.int32), unroll=True)
    return p ^ _INT_MIN


def _k1_body(x_ref, sg_ref):
    sg_ref[...] = jnp.max(_sortable(x_ref[...]), axis=1)


def _k1(x3, rpb):
    n, w, g = x3.shape
    return pl.pallas_call(
        _k1_body,
        grid=(n // rpb,),
        in_specs=[pl.BlockSpec((rpb, w, g), lambda i: (i, 0, 0))],
        out_specs=pl.BlockSpec((rpb, g), lambda i: (i, 0)),
        out_shape=jax.ShapeDtypeStruct((n, g), jnp.int32),
    )(x3)


def _k2_body(sg_ref, rank_ref, *, k):
    sg = sg_ref[...]
    n, g = sg.shape
    vg = _radix_kth(sg, k, axes=(1,))
    gt = sg > vg
    eq = sg == vg
    gtf = gt.astype(jnp.float32)
    eqf = eq.astype(jnp.float32)
    ia = jax.lax.broadcasted_iota(jnp.int32, (g, g), 0)
    ib = jax.lax.broadcasted_iota(jnp.int32, (g, g), 1)
    tri = (ia <= ib).astype(jnp.float32)
    cum_gt = jnp.dot(gtf, tri, preferred_element_type=jnp.float32)
    cum_eq = jnp.dot(eqf, tri, preferred_element_type=jnp.float32)
    cgt = jnp.sum(gtf, axis=1, keepdims=True)
    rgt = cum_gt - gtf
    req = cum_eq - eqf + cgt
    rank_ref[...] = jnp.where(gt, rgt, jnp.where(eq, req, np.float32(1e9)))


def _k2(sg, k):
    n, g = sg.shape
    return pl.pallas_call(
        functools.partial(_k2_body, k=k),
        grid=(1,),
        in_specs=[pl.BlockSpec((n, g), lambda i: (0, 0))],
        out_specs=pl.BlockSpec((n, g), lambda i: (0, 0)),
        out_shape=jax.ShapeDtypeStruct((n, g), jnp.float32),
    )(sg)


def _k3_body(x_ref, rank_ref, c_ref, *, k):
    x = x_ref[...]
    rank = rank_ref[...]
    slots = jax.lax.broadcasted_iota(jnp.int32, (1, k, 1), 1).astype(jnp.float32)
    sel = (rank[:, None, :] == slots).astype(jnp.float32)
    c_ref[...] = jax.lax.dot_general(
        sel, x, (((2,), (2,)), ((0,), (0,))),
        precision=jax.lax.Precision.HIGHEST,
        preferred_element_type=jnp.float32)


def _k3(x3, rank, k, rpb):
    n, w, g = x3.shape
    return pl.pallas_call(
        functools.partial(_k3_body, k=k),
        grid=(n // rpb,),
        in_specs=[pl.BlockSpec((rpb, w, g), lambda i: (i, 0, 0)),
                  pl.BlockSpec((rpb, g), lambda i: (i, 0))],
        out_specs=pl.BlockSpec((rpb, k, w), lambda i: (i, 0, 0)),
        out_shape=jax.ShapeDtypeStruct((n, k, w), jnp.float32),
    )(x3, rank)


def _k4_body(c_ref, v_ref, *, k):
    s = _sortable(c_ref[...])
    v_ref[...] = _radix_kth(s, k, axes=(1, 2))[:, :, 0]


def _k4(compact, k, rpb):
    n, kk, w = compact.shape
    return pl.pallas_call(
        functools.partial(_k4_body, k=k),
        grid=(n // rpb,),
        in_specs=[pl.BlockSpec((rpb, kk, w), lambda i: (i, 0, 0))],
        out_specs=pl.BlockSpec((rpb, 1), lambda i: (i, 0)),
        out_shape=jax.ShapeDtypeStruct((n, 1), jnp.int32),
    )(compact)


def _k5_body(x_ref, v_ref, o_ref):
    x = x_ref[...]
    v = v_ref[...][:, :, None]
    s = _sortable(x)
    o_ref[...] = jnp.where(s >= v, jnp.maximum(x, 0.0), 0.0)


def _k5(x3, v, rpb):
    n, w, g = x3.shape
    return pl.pallas_call(
        _k5_body,
        grid=(n // rpb,),
        in_specs=[pl.BlockSpec((rpb, w, g), lambda i: (i, 0, 0)),
                  pl.BlockSpec((rpb, 1), lambda i: (i, 0))],
        out_specs=pl.BlockSpec((rpb, w, g), lambda i: (i, 0, 0)),
        out_shape=jax.ShapeDtypeStruct((n, w, g), x3.dtype),
    )(x3, v)


def kernel(features):
    b, l, d = features.shape
    n = b * l
    x3 = features.reshape(n, _W, d // _W)
    sg = _k1(x3, 64)
    rank = _k2(sg, _K)
    compact = _k3(x3, rank, _K, 32)
    v = _k4(compact, _K, 512)
    out = _k5(x3, v, 64)
    return out.reshape(b, l, d)


# speedup vs baseline: 1.1255x; 1.1252x over previous
import functools

import jax
import jax.numpy as jnp
import numpy as np
from jax.experimental import pallas as pl

_K = 64
_W = 64
_INT_MIN = np.int32(-(2 ** 31))
_TOP_MASK = np.int32(0x7FFFFFFF)


def _sortable(x):
    xi = jax.lax.bitcast_convert_type(x, jnp.int32)
    return xi ^ (jax.lax.shift_right_arithmetic(xi, 31) & _TOP_MASK)


def _radix_kth(s, k, axes):
    def body(i, p):
        bit = jax.lax.shift_left(np.int32(1), (31 - i).astype(jnp.int32))
        cand = p | bit
        thr = cand ^ _INT_MIN
        cnt = jnp.sum((s >= thr).astype(jnp.int32), axis=axes, keepdims=True)
        return jnp.where(cnt >= k, cand, p)

    shape = tuple(1 if d in axes else n for d, n in enumerate(s.shape))
    p = jax.lax.fori_loop(0, 32, body, jnp.zeros(shape, jnp.int32), unroll=True)
    return p ^ _INT_MIN


def _grp_max(s, w, g):
    parts = [s[..., i * g:(i + 1) * g] for i in range(w)]
    while len(parts) > 1:
        parts = [jnp.maximum(parts[i], parts[i + 1])
                 for i in range(0, len(parts), 2)]
    return parts[0]


def _k1_body(x_ref, sg_ref, *, w, g):
    sg_ref[...] = _grp_max(_sortable(x_ref[...]), w, g)


def _k1(x, w, g, rpb):
    b, l, d = x.shape
    return pl.pallas_call(
        functools.partial(_k1_body, w=w, g=g),
        grid=(b // rpb,),
        in_specs=[pl.BlockSpec((rpb, l, d), lambda i: (i, 0, 0))],
        out_specs=pl.BlockSpec((rpb, l, g), lambda i: (i, 0, 0)),
        out_shape=jax.ShapeDtypeStruct((b, l, g), jnp.int32),
    )(x)


def _k2_body(sg_ref, rank_ref, *, k):
    sg = sg_ref[...]
    g = sg.shape[-1]
    vg = _radix_kth(sg, k, axes=(2,))
    gt = sg > vg
    eq = sg == vg
    gtf = gt.astype(jnp.float32)
    eqf = eq.astype(jnp.float32)
    ia = jax.lax.broadcasted_iota(jnp.int32, (g, g), 0)
    ib = jax.lax.broadcasted_iota(jnp.int32, (g, g), 1)
    tri = (ia <= ib).astype(jnp.float32)
    cum_gt = jax.lax.dot_general(gtf, tri, (((2,), (0,)), ((), ())),
                                 preferred_element_type=jnp.float32)
    cum_eq = jax.lax.dot_general(eqf, tri, (((2,), (0,)), ((), ())),
                                 preferred_element_type=jnp.float32)
    cgt = jnp.sum(gtf, axis=2, keepdims=True)
    rgt = cum_gt - gtf
    req = cum_eq - eqf + cgt
    rank_ref[...] = jnp.where(gt, rgt, jnp.where(eq, req, np.float32(1e9)))


def _k2(sg, k):
    b, l, g = sg.shape
    return pl.pallas_call(
        functools.partial(_k2_body, k=k),
        grid=(1,),
        in_specs=[pl.BlockSpec((b, l, g), lambda i: (0, 0, 0))],
        out_specs=pl.BlockSpec((b, l, g), lambda i: (0, 0, 0)),
        out_shape=jax.ShapeDtypeStruct((b, l, g), jnp.float32),
    )(sg)


def _k3_body(x_ref, rank_ref, c_ref, *, k, w, g):
    x = x_ref[...][0]
    rank = rank_ref[...][0]
    xs = jnp.stack([x[:, i * g:(i + 1) * g] for i in range(w)], axis=1)
    slots = jax.lax.broadcasted_iota(jnp.int32, (1, k, 1), 1).astype(jnp.float32)
    sel = (rank[:, None, :] == slots).astype(jnp.float32)
    c = jax.lax.dot_general(
        sel, xs, (((2,), (2,)), ((0,), (0,))),
        precision=jax.lax.Precision.HIGHEST,
        preferred_element_type=jnp.float32)
    c_ref[...] = c[None]


def _k3(x, rank, k, w, g):
    b, l, d = x.shape
    return pl.pallas_call(
        functools.partial(_k3_body, k=k, w=w, g=g),
        grid=(b,),
        in_specs=[pl.BlockSpec((1, l, d), lambda i: (i, 0, 0)),
                  pl.BlockSpec((1, l, g), lambda i: (i, 0, 0))],
        out_specs=pl.BlockSpec((1, l, k, w), lambda i: (i, 0, 0, 0)),
        out_shape=jax.ShapeDtypeStruct((b, l, k, w), jnp.float32),
    )(x, rank)


def _k4_body(c_ref, v_ref, *, k):
    s = _sortable(c_ref[...])
    v_ref[...] = _radix_kth(s, k, axes=(2, 3))[:, :, :, 0]


def _k4(compact, k, rpb):
    b, l, kk, w = compact.shape
    return pl.pallas_call(
        functools.partial(_k4_body, k=k),
        grid=(b // rpb,),
        in_specs=[pl.BlockSpec((rpb, l, kk, w), lambda i: (i, 0, 0, 0))],
        out_specs=pl.BlockSpec((rpb, l, 1), lambda i: (i, 0, 0)),
        out_shape=jax.ShapeDtypeStruct((b, l, 1), jnp.int32),
    )(compact)


def _k5_body(x_ref, v_ref, o_ref):
    x = x_ref[...]
    v = v_ref[...]
    s = _sortable(x)
    o_ref[...] = jnp.where(s >= v, jnp.maximum(x, 0.0), 0.0)


def _k5(x, v, rpb):
    b, l, d = x.shape
    return pl.pallas_call(
        _k5_body,
        grid=(b // rpb,),
        in_specs=[pl.BlockSpec((rpb, l, d), lambda i: (i, 0, 0)),
                  pl.BlockSpec((rpb, l, 1), lambda i: (i, 0, 0))],
        out_specs=pl.BlockSpec((rpb, l, d), lambda i: (i, 0, 0)),
        out_shape=jax.ShapeDtypeStruct((b, l, d), x.dtype),
    )(x, v)


def kernel(features):
    b, l, d = features.shape
    g = d // _W
    sg = _k1(features, _W, g, 4)
    rank = _k2(sg, _K)
    compact = _k3(features, rank, _K, _W, g)
    v = _k4(compact, _K, 32)
    out = _k5(features, v, 4)
    return out
